# baseline (speedup 1.0000x reference)
import dataclasses
import functools

import jax
import jax.numpy as jnp
from jax import lax
from jax.experimental import pallas as pl
from jax.experimental.pallas import tpu as pltpu
from jax.experimental.pallas import tpu_sc as plsc

NCORE = 2
NSUB = 16
NW = NCORE * NSUB
BLK = 128
ZR = 64


def _sc_compiler_params(tc_tiling=True):
    cp = pltpu.CompilerParams()
    if "needs_layout_passes" in pltpu.CompilerParams.__dataclass_fields__:
        cp = dataclasses.replace(cp, needs_layout_passes=False)
    if not tc_tiling:
        cp = dataclasses.replace(cp, use_tc_tiling_on_sc=False)
    return cp


def _pad_counts(n_nodes, n_edges):
    align = NSUB * ZR
    np_ = ((n_nodes + align) // align) * align
    nb = -(-n_edges // (NW * BLK))
    nb += nb % 2
    return np_, nb


@functools.cache
def _deg_kernel(np_, nb):
    mesh = plsc.VectorSubcoreMesh(core_axis_name="c", subcore_axis_name="s")

    @functools.partial(
        pl.kernel,
        out_type=jax.ShapeDtypeStruct((2, NW, np_), jnp.float32),
        mesh=mesh,
        compiler_params=_sc_compiler_params(),
        scratch_types=[
            pltpu.VMEM((nb, BLK), jnp.int32),
            pltpu.VMEM((nb, BLK), jnp.int32),
            pltpu.VMEM((np_,), jnp.float32),
            pltpu.VMEM((np_,), jnp.float32),
        ],
    )
    def deg(src_hbm, dst_hbm, znp_hbm, out_hbm, sidx, didx, hsrc, hdst):
        c = lax.axis_index("c")
        s = lax.axis_index("s")
        w = c * NSUB + s
        pltpu.sync_copy(src_hbm.at[w], sidx)
        pltpu.sync_copy(dst_hbm.at[w], didx)
        pltpu.sync_copy(znp_hbm, hsrc)
        pltpu.sync_copy(znp_hbm, hdst)
        ones = jnp.full((16,), 1.0, jnp.float32)

        @pl.loop(0, nb)
        def _(j):
            for l in range(BLK // 16):
                sv = sidx[j, pl.ds(l * 16, 16)]
                dv = didx[j, pl.ds(l * 16, 16)]
                plsc.addupdate_scatter(hsrc, [sv], ones)
                plsc.addupdate_scatter(hdst, [dv], ones)

        pltpu.sync_copy(hsrc, out_hbm.at[0, w])
        pltpu.sync_copy(hdst, out_hbm.at[1, w])

    return deg


@functools.cache
def _agg_kernel(np_, nb, d):
    mesh = plsc.VectorSubcoreMesh(core_axis_name="c", subcore_axis_name="s")
    rows_per_tile = np_ // NSUB
    dh = d // 2

    @functools.partial(
        pl.kernel,
        out_type=jax.ShapeDtypeStruct((NCORE, 2, np_, dh), jnp.float32),
        mesh=mesh,
        compiler_params=_sc_compiler_params(tc_tiling=False),
        scratch_types=[
            pltpu.VMEM((nb, BLK), jnp.int32),
            pltpu.VMEM((nb, BLK), jnp.int32),
            pltpu.VMEM((BLK, dh), jnp.float32),
            pltpu.VMEM((BLK, dh), jnp.float32),
            pltpu.VMEM((ZR, dh), jnp.float32),
            pltpu.VMEM_SHARED((np_, dh), jnp.float32),
            pltpu.SemaphoreType.DMA,
            pltpu.SemaphoreType.DMA,
            pltpu.SemaphoreType.DMA,
            pltpu.SemaphoreType.DMA,
        ],
    )
    def agg(mlo_hbm, mhi_hbm, src_hbm, dst_hbm, z_hbm, out_hbm, sidx, didx,
            g0, g1, zbuf, acc, gs0, gs1, ss0, ss1):
        c = lax.axis_index("c")
        s = lax.axis_index("s")
        w = c * NSUB + s
        pltpu.sync_copy(src_hbm.at[w], sidx)
        pltpu.sync_copy(dst_hbm.at[w], didx)
        pltpu.sync_copy(z_hbm, zbuf)
        base = s * rows_per_tile

        for h, m_hbm in enumerate((mlo_hbm, mhi_hbm)):

            @pl.loop(0, rows_per_tile // ZR)
            def _(k):
                pltpu.sync_copy(zbuf, acc.at[pl.ds(base + k * ZR, ZR)])

            plsc.subcore_barrier()

            def gstart(j, buf, sem):
                pltpu.async_copy(m_hbm.at[sidx.at[j]], buf, sem)

            def gwait(j, buf, sem):
                pltpu.make_async_copy(m_hbm.at[sidx.at[j]], buf, sem).wait()

            def sstart(j, buf, sem):
                pltpu.async_copy(buf, acc.at[didx.at[j]], sem, add=True)

            def swait(j, buf, sem):
                pltpu.make_async_copy(buf, acc.at[didx.at[j]], sem).wait()

            gstart(0, g0, gs0)
            gstart(1, g1, gs1)

            @pl.loop(0, nb - 2, step=2)
            def _(j):
                gwait(j, g0, gs0)
                sstart(j, g0, ss0)
                gwait(j + 1, g1, gs1)
                sstart(j + 1, g1, ss1)
                swait(j, g0, ss0)
                gstart(j + 2, g0, gs0)
                swait(j + 1, g1, ss1)
                gstart(j + 3, g1, gs1)

            gwait(nb - 2, g0, gs0)
            sstart(nb - 2, g0, ss0)
            gwait(nb - 1, g1, gs1)
            sstart(nb - 1, g1, ss1)
            swait(nb - 2, g0, ss0)
            swait(nb - 1, g1, ss1)

            plsc.subcore_barrier()

            @pl.loop(0, rows_per_tile // ZR)
            def _(k):
                r = base + k * ZR
                pltpu.sync_copy(acc.at[pl.ds(r, ZR)],
                                out_hbm.at[c, h, pl.ds(r, ZR)])

            plsc.subcore_barrier()

    return agg


def _norms_m1(xp, degp, w1):
    np_ = xp.shape[0]
    d_hid = w1.shape[1]

    def body(x_ref, dp_ref, w_ref, mlo_ref, mhi_ref, ns_ref, nd_ref):
        dp = dp_ref[...]
        ones = jnp.ones((NW, 1), jnp.float32)
        cn = (((0,), (0,)), ((), ()))
        degs = lax.dot_general(dp[0], ones, cn,
                               preferred_element_type=jnp.float32)
        degd = lax.dot_general(dp[1], ones, cn,
                               preferred_element_type=jnp.float32)
        ns = jnp.where(degs > 0, lax.rsqrt(jnp.maximum(degs, 1.0)), 0.0)
        nd = jnp.where(degd > 0, lax.rsqrt(jnp.maximum(degd, 1.0)), 0.0)
        ns_ref[...] = ns
        nd_ref[...] = nd
        m = jnp.dot(x_ref[...] * ns, w_ref[...],
                    preferred_element_type=jnp.float32)
        mlo_ref[...] = m[:, : d_hid // 2]
        mhi_ref[...] = m[:, d_hid // 2:]

    return pl.pallas_call(
        body,
        out_shape=(
            jax.ShapeDtypeStruct((np_, d_hid // 2), jnp.float32),
            jax.ShapeDtypeStruct((np_, d_hid // 2), jnp.float32),
            jax.ShapeDtypeStruct((np_, 1), jnp.float32),
            jax.ShapeDtypeStruct((np_, 1), jnp.float32),
        ),
    )(xp, degp, w1)


def _mid_layer(p, ns, nd, b1, w2):
    np_ = p.shape[2]
    d_out = w2.shape[1]

    def body(p_ref, ns_ref, nd_ref, b_ref, w_ref, mlo_ref, mhi_ref):
        agg = jnp.concatenate(
            [p_ref[0, 0] + p_ref[1, 0], p_ref[0, 1] + p_ref[1, 1]], axis=1)
        h = jnp.maximum(agg * nd_ref[...] + b_ref[...], 0.0)
        m = jnp.dot(h * ns_ref[...], w_ref[...],
                    preferred_element_type=jnp.float32)
        mlo_ref[...] = m[:, : d_out // 2]
        mhi_ref[...] = m[:, d_out // 2:]

    return pl.pallas_call(
        body,
        out_shape=(
            jax.ShapeDtypeStruct((np_, d_out // 2), jnp.float32),
            jax.ShapeDtypeStruct((np_, d_out // 2), jnp.float32),
        ),
    )(p, ns, nd, b1, w2)


def _head(p, nd, b2, wm1, bm1, wm2, bm2):
    np_ = p.shape[2]
    d_out = wm2.shape[1]

    def body(p_ref, nd_ref, b_ref, w1_ref, b1_ref, w2_ref, b2_ref, o_ref):
        agg = jnp.concatenate(
            [p_ref[0, 0] + p_ref[1, 0], p_ref[0, 1] + p_ref[1, 1]], axis=1)
        h = jnp.maximum(agg * nd_ref[...] + b_ref[...], 0.0)
        z = jnp.dot(h, w1_ref[...], preferred_element_type=jnp.float32)
        z = 1.0 / (1.0 + jnp.exp(-(z + b1_ref[...])))
        o_ref[...] = (jnp.dot(z, w2_ref[...],
                              preferred_element_type=jnp.float32) + b2_ref[...])

    return pl.pallas_call(
        body,
        out_shape=jax.ShapeDtypeStruct((np_, d_out), jnp.float32),
    )(p, nd, b2, wm1, bm1, wm2, bm2)


def kernel(x, edge_index, W1, b1, W2, b2, Wm1, bm1, Wm2, bm2):
    n, d_in = x.shape
    e = edge_index.shape[1]
    np_, nb = _pad_counts(n, e)
    ep = NW * nb * BLK

    pad_idx = n + (jnp.arange(ep - e, dtype=jnp.int32) % (np_ - n))
    srcp = jnp.concatenate([edge_index[0], pad_idx]).reshape(NW, nb, BLK)
    dstp = jnp.concatenate([edge_index[1], pad_idx]).reshape(NW, nb, BLK)
    xp = jnp.pad(x, ((0, np_ - n), (0, 0)))
    znp = jnp.zeros((np_,), jnp.float32)
    zbuf = jnp.zeros((ZR, d_in // 2), jnp.float32)

    degp = _deg_kernel(np_, nb)(srcp, dstp, znp)
    m1lo, m1hi, ns, nd = _norms_m1(xp, degp, W1)
    p1 = _agg_kernel(np_, nb, W1.shape[1])(m1lo, m1hi, srcp, dstp, zbuf)
    m2lo, m2hi = _mid_layer(p1, ns, nd, b1.reshape(1, -1), W2)
    p2 = _agg_kernel(np_, nb, W2.shape[1])(m2lo, m2hi, srcp, dstp, zbuf)
    out = _head(p2, nd, b2.reshape(1, -1), Wm1, bm1.reshape(1, -1),
                Wm2, bm2.reshape(1, -1))
    return out[:n]


# speedup vs baseline: 1.1295x; 1.1295x over previous
import dataclasses
import functools

import jax
import jax.numpy as jnp
from jax import lax
from jax.experimental import pallas as pl
from jax.experimental.pallas import tpu as pltpu
from jax.experimental.pallas import tpu_sc as plsc

NCORE = 2
NSUB = 16
NW = NCORE * NSUB
BLK = 128
ZR = 64


def _sc_compiler_params():
    cp = pltpu.CompilerParams()
    if "needs_layout_passes" in pltpu.CompilerParams.__dataclass_fields__:
        cp = dataclasses.replace(cp, needs_layout_passes=False)
    return cp


def _pad_counts(n_nodes, n_edges):
    align = NSUB * ZR
    np_ = ((n_nodes + align) // align) * align
    nb = -(-n_edges // (NW * BLK))
    return np_, nb


@functools.cache
def _deg_kernel(np_, nb):
    mesh = plsc.VectorSubcoreMesh(core_axis_name="c", subcore_axis_name="s")

    @functools.partial(
        pl.kernel,
        out_type=jax.ShapeDtypeStruct((2, NW, np_), jnp.float32),
        mesh=mesh,
        compiler_params=_sc_compiler_params(),
        scratch_types=[
            pltpu.VMEM((nb, BLK), jnp.int32),
            pltpu.VMEM((nb, BLK), jnp.int32),
            pltpu.VMEM((np_,), jnp.float32),
            pltpu.VMEM((np_,), jnp.float32),
        ],
    )
    def deg(src_hbm, dst_hbm, znp_hbm, out_hbm, sidx, didx, hsrc, hdst):
        c = lax.axis_index("c")
        s = lax.axis_index("s")
        w = c * NSUB + s
        pltpu.sync_copy(src_hbm.at[w], sidx)
        pltpu.sync_copy(dst_hbm.at[w], didx)
        pltpu.sync_copy(znp_hbm, hsrc)
        pltpu.sync_copy(znp_hbm, hdst)
        ones = jnp.full((16,), 1.0, jnp.float32)

        @pl.loop(0, nb)
        def _(j):
            for l in range(BLK // 16):
                sv = sidx[j, pl.ds(l * 16, 16)]
                dv = didx[j, pl.ds(l * 16, 16)]
                plsc.addupdate_scatter(hsrc, [sv], ones)
                plsc.addupdate_scatter(hdst, [dv], ones)

        pltpu.sync_copy(hsrc, out_hbm.at[0, w])
        pltpu.sync_copy(hdst, out_hbm.at[1, w])

    return deg


@functools.cache
def _agg_kernel(np_, nb, d):
    mesh = plsc.VectorSubcoreMesh(core_axis_name="c", subcore_axis_name="s")
    rows_per_tile = np_ // NSUB

    @functools.partial(
        pl.kernel,
        out_type=jax.ShapeDtypeStruct((NCORE, np_, d), jnp.float32),
        mesh=mesh,
        compiler_params=_sc_compiler_params(),
        scratch_types=[
            pltpu.VMEM((nb, BLK), jnp.int32),
            pltpu.VMEM((nb, BLK), jnp.int32),
            pltpu.VMEM((BLK, d), jnp.float32),
            pltpu.VMEM((ZR, d), jnp.float32),
            pltpu.VMEM_SHARED((np_, d), jnp.float32),
        ],
    )
    def agg(m_hbm, src_hbm, dst_hbm, z_hbm, out_hbm, sidx, didx, msgs, zbuf,
            acc):
        c = lax.axis_index("c")
        s = lax.axis_index("s")
        w = c * NSUB + s
        pltpu.sync_copy(src_hbm.at[w], sidx)
        pltpu.sync_copy(dst_hbm.at[w], didx)
        pltpu.sync_copy(z_hbm, zbuf)
        base = s * rows_per_tile

        @pl.loop(0, rows_per_tile // ZR)
        def _(k):
            pltpu.sync_copy(zbuf, acc.at[pl.ds(base + k * ZR, ZR)])

        plsc.subcore_barrier()

        @pl.when(lax.rem(s, 2) == 1)
        def _():
            pltpu.sync_copy(m_hbm.at[sidx.at[0]], msgs)

        @pl.loop(0, nb)
        def _(j):
            pltpu.sync_copy(m_hbm.at[sidx.at[j]], msgs)
            pltpu.sync_copy(msgs, acc.at[didx.at[j]], add=True)

        plsc.subcore_barrier()

        @pl.loop(0, rows_per_tile // ZR)
        def _(k):
            r = base + k * ZR
            pltpu.sync_copy(acc.at[pl.ds(r, ZR)], out_hbm.at[c, pl.ds(r, ZR)])

    return agg


def _norms_m1(xp, degp, w1):
    np_ = xp.shape[0]
    d_hid = w1.shape[1]

    def body(x_ref, dp_ref, w_ref, m_ref, ns_ref, nd_ref):
        dp = dp_ref[...]
        ones = jnp.ones((NW, 1), jnp.float32)
        cn = (((0,), (0,)), ((), ()))
        degs = lax.dot_general(dp[0], ones, cn,
                               preferred_element_type=jnp.float32)
        degd = lax.dot_general(dp[1], ones, cn,
                               preferred_element_type=jnp.float32)
        ns = jnp.where(degs > 0, lax.rsqrt(jnp.maximum(degs, 1.0)), 0.0)
        nd = jnp.where(degd > 0, lax.rsqrt(jnp.maximum(degd, 1.0)), 0.0)
        ns_ref[...] = ns
        nd_ref[...] = nd
        m_ref[...] = jnp.dot(x_ref[...] * ns, w_ref[...],
                             preferred_element_type=jnp.float32)

    return pl.pallas_call(
        body,
        out_shape=(
            jax.ShapeDtypeStruct((np_, d_hid), jnp.float32),
            jax.ShapeDtypeStruct((np_, 1), jnp.float32),
            jax.ShapeDtypeStruct((np_, 1), jnp.float32),
        ),
    )(xp, degp, w1)


def _mid_layer(p, ns, nd, b1, w2):
    np_ = p.shape[1]
    d_out = w2.shape[1]

    def body(p_ref, ns_ref, nd_ref, b_ref, w_ref, m_ref):
        agg = p_ref[0] + p_ref[1]
        h = jnp.maximum(agg * nd_ref[...] + b_ref[...], 0.0)
        m_ref[...] = jnp.dot(h * ns_ref[...], w_ref[...],
                             preferred_element_type=jnp.float32)

    return pl.pallas_call(
        body,
        out_shape=jax.ShapeDtypeStruct((np_, d_out), jnp.float32),
    )(p, ns, nd, b1, w2)


def _head(p, nd, b2, wm1, bm1, wm2, bm2, n):
    d_out = wm2.shape[1]

    def body(p_ref, nd_ref, b_ref, w1_ref, b1_ref, w2_ref, b2_ref, o_ref):
        agg = p_ref[0, :n] + p_ref[1, :n]
        h = jnp.maximum(agg * nd_ref[:n] + b_ref[...], 0.0)
        z = jnp.dot(h, w1_ref[...], preferred_element_type=jnp.float32)
        z = 1.0 / (1.0 + jnp.exp(-(z + b1_ref[...])))
        o_ref[...] = (jnp.dot(z, w2_ref[...],
                              preferred_element_type=jnp.float32) + b2_ref[...])

    return pl.pallas_call(
        body,
        out_shape=jax.ShapeDtypeStruct((n, d_out), jnp.float32),
    )(p, nd, b2, wm1, bm1, wm2, bm2)


def kernel(x, edge_index, W1, b1, W2, b2, Wm1, bm1, Wm2, bm2):
    n, d_in = x.shape
    e = edge_index.shape[1]
    np_, nb = _pad_counts(n, e)
    ep = NW * nb * BLK

    pad_idx = n + (jnp.arange(ep - e, dtype=jnp.int32) % (np_ - n))
    srcp = jnp.concatenate([edge_index[0], pad_idx]).reshape(NW, nb, BLK)
    dstp = jnp.concatenate([edge_index[1], pad_idx]).reshape(NW, nb, BLK)
    xp = jnp.pad(x, ((0, np_ - n), (0, 0)))
    znp = jnp.zeros((np_,), jnp.float32)
    zbuf = jnp.zeros((ZR, d_in), jnp.float32)

    degp = _deg_kernel(np_, nb)(srcp, dstp, znp)
    m1, ns, nd = _norms_m1(xp, degp, W1)
    p1 = _agg_kernel(np_, nb, W1.shape[1])(m1, srcp, dstp, zbuf)
    m2 = _mid_layer(p1, ns, nd, b1.reshape(1, -1), W2)
    p2 = _agg_kernel(np_, nb, W2.shape[1])(m2, srcp, dstp, zbuf)
    return _head(p2, nd, b2.reshape(1, -1), Wm1, bm1.reshape(1, -1),
                 Wm2, bm2.reshape(1, -1), n)


# speedup vs baseline: 1.6425x; 1.4542x over previous
import dataclasses
import functools

import jax
import jax.numpy as jnp
from jax import lax
from jax.experimental import pallas as pl
from jax.experimental.pallas import tpu as pltpu
from jax.experimental.pallas import tpu_sc as plsc

NCORE = 2
NSUB = 16
NW = NCORE * NSUB
BLK = 128
ZR = 64
CH = 40


def _sc_compiler_params():
    cp = pltpu.CompilerParams()
    if "needs_layout_passes" in pltpu.CompilerParams.__dataclass_fields__:
        cp = dataclasses.replace(cp, needs_layout_passes=False)
    return cp


def _pad_counts(n_nodes, n_edges):
    align = NSUB * ZR
    np_ = ((n_nodes + align) // align) * align
    if np_ - n_nodes < BLK:
        np_ += align
    nb = -(-n_edges // (NW * BLK))
    nb = -(-nb // (2 * CH)) * 2 * CH
    return np_, nb


@functools.cache
def _deg_kernel(np_, nb):
    mesh = plsc.VectorSubcoreMesh(core_axis_name="c", subcore_axis_name="s")

    @functools.partial(
        pl.kernel,
        out_type=jax.ShapeDtypeStruct((2, NW, np_), jnp.float32),
        mesh=mesh,
        compiler_params=_sc_compiler_params(),
        scratch_types=[
            pltpu.VMEM((nb, BLK), jnp.int32),
            pltpu.VMEM((nb, BLK), jnp.int32),
            pltpu.VMEM((np_,), jnp.float32),
            pltpu.VMEM((np_,), jnp.float32),
        ],
    )
    def deg(src_hbm, dst_hbm, znp_hbm, out_hbm, sidx, didx, hsrc, hdst):
        c = lax.axis_index("c")
        s = lax.axis_index("s")
        w = c * NSUB + s
        pltpu.sync_copy(src_hbm.at[w], sidx)
        pltpu.sync_copy(dst_hbm.at[w], didx)
        pltpu.sync_copy(znp_hbm, hsrc)
        pltpu.sync_copy(znp_hbm, hdst)
        ones = jnp.full((16,), 1.0, jnp.float32)

        @pl.loop(0, nb)
        def _(j):
            for l in range(BLK // 16):
                sv = sidx[j, pl.ds(l * 16, 16)]
                dv = didx[j, pl.ds(l * 16, 16)]
                plsc.addupdate_scatter(hsrc, [sv], ones)
                plsc.addupdate_scatter(hdst, [dv], ones)

        pltpu.sync_copy(hsrc, out_hbm.at[0, w])
        pltpu.sync_copy(hdst, out_hbm.at[1, w])

    return deg


@functools.cache
def _agg_kernel(np_, nb, d, n):
    mesh = plsc.VectorSubcoreMesh(core_axis_name="c", subcore_axis_name="s")
    rows_per_tile = np_ // NSUB

    @functools.partial(
        pl.kernel,
        out_type=jax.ShapeDtypeStruct((NCORE, np_, d), jnp.float32),
        mesh=mesh,
        compiler_params=_sc_compiler_params(),
        scratch_types=[
            pltpu.VMEM((nb, BLK), jnp.int32),
            pltpu.VMEM((CH, BLK), jnp.int32),
            pltpu.VMEM((BLK, d), jnp.float32),
            pltpu.VMEM((BLK, d), jnp.float32),
            pltpu.VMEM((1, BLK), jnp.int32),
            pltpu.VMEM_SHARED((np_, d), jnp.float32),
            pltpu.SemaphoreType.DMA,
            pltpu.SemaphoreType.DMA,
        ],
    )
    def agg(m_hbm, src_hbm, dst_hbm, out_hbm, sidx, didx, g0, g1, piota,
            acc, s0, s1):
        c = lax.axis_index("c")
        s = lax.axis_index("s")
        w = c * NSUB + s
        pltpu.sync_copy(src_hbm.at[w], sidx)
        pltpu.sync_copy(dst_hbm.at[w, pl.ds(0, CH)], didx)
        base = s * rows_per_tile

        start = n + s * ((np_ - n - BLK) // NSUB)
        for l in range(BLK // 16):
            piota[0, pl.ds(l * 16, 16)] = (
                start + l * 16 + lax.iota(jnp.int32, 16))
        pltpu.sync_copy(m_hbm.at[piota.at[0]], g0)

        @pl.loop(0, rows_per_tile // BLK)
        def _(k):
            pltpu.sync_copy(g0, acc.at[pl.ds(base + k * BLK, BLK)])

        plsc.subcore_barrier()

        def gstart(j, buf, sem):
            pltpu.async_copy(m_hbm.at[sidx.at[j]], buf, sem)

        def gwait(j, buf, sem):
            pltpu.make_async_copy(m_hbm.at[sidx.at[j]], buf, sem).wait()

        def scat(r, buf):
            pltpu.sync_copy(buf, acc.at[didx.at[r]], add=True)

        gstart(0, g0, s0)
        gstart(1, g1, s1)

        @pl.loop(0, nb - 2, step=2)
        def _(j):
            r = lax.rem(j, CH)

            @pl.when(jnp.logical_and(r == 0, j > 0))
            def _():
                pltpu.sync_copy(
                    dst_hbm.at[w, pl.ds(pl.multiple_of(j, CH), CH)], didx)

            gwait(j, g0, s0)
            scat(r, g0)
            gstart(j + 2, g0, s0)
            gwait(j + 1, g1, s1)
            scat(r + 1, g1)

            @pl.when(j + 3 < nb)
            def _():
                gstart(j + 3, g1, s1)

        gwait(nb - 2, g0, s0)
        scat(lax.rem(nb - 2, CH), g0)
        gwait(nb - 1, g1, s1)
        scat(lax.rem(nb - 1, CH), g1)

        plsc.subcore_barrier()

        @pl.loop(0, rows_per_tile // BLK)
        def _(k):
            r = base + k * BLK
            pltpu.sync_copy(acc.at[pl.ds(r, BLK)], out_hbm.at[c, pl.ds(r, BLK)])

    return agg


def _norms_m1(xp, degp, w1):
    np_ = xp.shape[0]
    d_hid = w1.shape[1]

    def body(x_ref, dp_ref, w_ref, m_ref, ns_ref, nd_ref):
        dp = dp_ref[...]
        ones = jnp.ones((NW, 1), jnp.float32)
        cn = (((0,), (0,)), ((), ()))
        degs = lax.dot_general(dp[0], ones, cn,
                               preferred_element_type=jnp.float32)
        degd = lax.dot_general(dp[1], ones, cn,
                               preferred_element_type=jnp.float32)
        ns = jnp.where(degs > 0, lax.rsqrt(jnp.maximum(degs, 1.0)), 0.0)
        nd = jnp.where(degd > 0, lax.rsqrt(jnp.maximum(degd, 1.0)), 0.0)
        ns_ref[...] = ns
        nd_ref[...] = nd
        m_ref[...] = jnp.dot(x_ref[...] * ns, w_ref[...],
                             preferred_element_type=jnp.float32)

    return pl.pallas_call(
        body,
        out_shape=(
            jax.ShapeDtypeStruct((np_, d_hid), jnp.float32),
            jax.ShapeDtypeStruct((np_, 1), jnp.float32),
            jax.ShapeDtypeStruct((np_, 1), jnp.float32),
        ),
    )(xp, degp, w1)


def _mid_layer(p, ns, nd, b1, w2):
    np_ = p.shape[1]
    d_out = w2.shape[1]

    def body(p_ref, ns_ref, nd_ref, b_ref, w_ref, m_ref):
        agg = p_ref[0] + p_ref[1]
        h = jnp.maximum(agg * nd_ref[...] + b_ref[...], 0.0)
        m_ref[...] = jnp.dot(h * ns_ref[...], w_ref[...],
                             preferred_element_type=jnp.float32)

    return pl.pallas_call(
        body,
        out_shape=jax.ShapeDtypeStruct((np_, d_out), jnp.float32),
    )(p, ns, nd, b1, w2)


def _head(p, nd, b2, wm1, bm1, wm2, bm2, n):
    d_out = wm2.shape[1]

    def body(p_ref, nd_ref, b_ref, w1_ref, b1_ref, w2_ref, b2_ref, o_ref):
        agg = p_ref[0, :n] + p_ref[1, :n]
        h = jnp.maximum(agg * nd_ref[:n] + b_ref[...], 0.0)
        z = jnp.dot(h, w1_ref[...], preferred_element_type=jnp.float32)
        z = 1.0 / (1.0 + jnp.exp(-(z + b1_ref[...])))
        o_ref[...] = (jnp.dot(z, w2_ref[...],
                              preferred_element_type=jnp.float32) + b2_ref[...])

    return pl.pallas_call(
        body,
        out_shape=jax.ShapeDtypeStruct((n, d_out), jnp.float32),
    )(p, nd, b2, wm1, bm1, wm2, bm2)


def kernel(x, edge_index, W1, b1, W2, b2, Wm1, bm1, Wm2, bm2):
    n, d_in = x.shape
    e = edge_index.shape[1]
    np_, nb = _pad_counts(n, e)
    ep = NW * nb * BLK

    pad_idx = n + (jnp.arange(ep - e, dtype=jnp.int32) % (np_ - n))
    srcp = jnp.concatenate([edge_index[0], pad_idx]).reshape(NW, nb, BLK)
    dstp = jnp.concatenate([edge_index[1], pad_idx]).reshape(NW, nb, BLK)
    xp = jnp.pad(x, ((0, np_ - n), (0, 0)))
    znp = jnp.zeros((np_,), jnp.float32)

    degp = _deg_kernel(np_, nb)(srcp, dstp, znp)
    m1, ns, nd = _norms_m1(xp, degp, W1)
    p1 = _agg_kernel(np_, nb, W1.shape[1], n)(m1, srcp, dstp)
    m2 = _mid_layer(p1, ns, nd, b1.reshape(1, -1), W2)
    p2 = _agg_kernel(np_, nb, W2.shape[1], n)(m2, srcp, dstp)
    return _head(p2, nd, b2.reshape(1, -1), Wm1, bm1.reshape(1, -1),
                 Wm2, bm2.reshape(1, -1), n)


# speedup vs baseline: 1.6514x; 1.0054x over previous
import dataclasses
import functools

import jax
import jax.numpy as jnp
from jax import lax
from jax.experimental import pallas as pl
from jax.experimental.pallas import tpu as pltpu
from jax.experimental.pallas import tpu_sc as plsc

NCORE = 2
NSUB = 16
NW = NCORE * NSUB
BLK = 128
ZR = 64
CH = 40


def _sc_compiler_params():
    cp = pltpu.CompilerParams()
    if "needs_layout_passes" in pltpu.CompilerParams.__dataclass_fields__:
        cp = dataclasses.replace(cp, needs_layout_passes=False)
    return cp


def _pad_counts(n_nodes, n_edges):
    align = NSUB * ZR
    np_ = ((n_nodes + align) // align) * align
    if np_ - n_nodes < BLK:
        np_ += align
    nb = -(-n_edges // (NW * BLK))
    nb = -(-nb // (2 * CH)) * 2 * CH
    return np_, nb


@functools.cache
def _deg_kernel(np_, nb):
    mesh = plsc.VectorSubcoreMesh(core_axis_name="c", subcore_axis_name="s")

    @functools.partial(
        pl.kernel,
        out_type=jax.ShapeDtypeStruct((2, NW, np_), jnp.float32),
        mesh=mesh,
        compiler_params=_sc_compiler_params(),
        scratch_types=[
            pltpu.VMEM((nb, BLK), jnp.int32),
            pltpu.VMEM((nb, BLK), jnp.int32),
            pltpu.VMEM((np_,), jnp.float32),
            pltpu.VMEM((np_,), jnp.float32),
        ],
    )
    def deg(src_hbm, dst_hbm, znp_hbm, out_hbm, sidx, didx, hsrc, hdst):
        c = lax.axis_index("c")
        s = lax.axis_index("s")
        w = c * NSUB + s
        pltpu.sync_copy(src_hbm.at[w], sidx)
        pltpu.sync_copy(dst_hbm.at[w], didx)
        pltpu.sync_copy(znp_hbm, hsrc)
        pltpu.sync_copy(znp_hbm, hdst)
        ones = jnp.full((16,), 1.0, jnp.float32)

        @pl.loop(0, nb)
        def _(j):
            for l in range(BLK // 16):
                sv = sidx[j, pl.ds(l * 16, 16)]
                dv = didx[j, pl.ds(l * 16, 16)]
                plsc.addupdate_scatter(hsrc, [sv], ones)
                plsc.addupdate_scatter(hdst, [dv], ones)

        pltpu.sync_copy(hsrc, out_hbm.at[0, w])
        pltpu.sync_copy(hdst, out_hbm.at[1, w])

    return deg


@functools.cache
def _agg_kernel(np_, nb, d, n):
    mesh = plsc.VectorSubcoreMesh(core_axis_name="c", subcore_axis_name="s")
    rows_per_tile = np_ // NSUB

    @functools.partial(
        pl.kernel,
        out_type=jax.ShapeDtypeStruct((NCORE, np_, d), jnp.float32),
        mesh=mesh,
        compiler_params=_sc_compiler_params(),
        scratch_types=[
            pltpu.VMEM((nb, BLK), jnp.int32),
            pltpu.VMEM((CH, BLK), jnp.int32),
            pltpu.VMEM((BLK, d), jnp.float32),
            pltpu.VMEM((BLK, d), jnp.float32),
            pltpu.VMEM((1, BLK), jnp.int32),
            pltpu.VMEM_SHARED((np_, d), jnp.float32),
            pltpu.SemaphoreType.DMA,
            pltpu.SemaphoreType.DMA,
            pltpu.SemaphoreType.DMA,
        ],
    )
    def agg(m_hbm, src_hbm, dst_hbm, out_hbm, sidx, didx, g0, g1, piota,
            acc, s0, s1, ws):
        c = lax.axis_index("c")
        s = lax.axis_index("s")
        w = c * NSUB + s
        pltpu.sync_copy(src_hbm.at[w], sidx)
        base = s * rows_per_tile

        def gstart(j, buf, sem):
            pltpu.async_copy(m_hbm.at[sidx.at[j]], buf, sem)

        def gwait(j, buf, sem):
            pltpu.make_async_copy(m_hbm.at[sidx.at[j]], buf, sem).wait()

        def scat(r, buf):
            pltpu.sync_copy(buf, acc.at[didx.at[r]], add=True)

        gstart(0, g0, s0)
        pltpu.sync_copy(dst_hbm.at[w, pl.ds(0, CH)], didx)

        start = n + s * ((np_ - n - BLK) // NSUB)
        for l in range(BLK // 16):
            piota[0, pl.ds(l * 16, 16)] = (
                start + l * 16 + lax.iota(jnp.int32, 16))
        pltpu.sync_copy(m_hbm.at[piota.at[0]], g1)

        @pl.loop(0, rows_per_tile // BLK)
        def _(k):
            pltpu.sync_copy(g1, acc.at[pl.ds(base + k * BLK, BLK)])

        plsc.subcore_barrier()

        gstart(1, g1, s1)

        @pl.loop(0, nb - 2, step=2)
        def _(j):
            r = lax.rem(j, CH)

            @pl.when(jnp.logical_and(r == 0, j > 0))
            def _():
                pltpu.sync_copy(
                    dst_hbm.at[w, pl.ds(pl.multiple_of(j, CH), CH)], didx)

            gwait(j, g0, s0)
            scat(r, g0)
            gstart(j + 2, g0, s0)
            gwait(j + 1, g1, s1)
            scat(r + 1, g1)

            @pl.when(j + 3 < nb)
            def _():
                gstart(j + 3, g1, s1)

        gwait(nb - 2, g0, s0)
        scat(lax.rem(nb - 2, CH), g0)
        gwait(nb - 1, g1, s1)
        scat(lax.rem(nb - 1, CH), g1)

        plsc.subcore_barrier()

        @pl.loop(0, rows_per_tile // BLK)
        def _(k):
            r = base + k * BLK
            pltpu.async_copy(acc.at[pl.ds(r, BLK)],
                             out_hbm.at[c, pl.ds(r, BLK)], ws)

        @pl.loop(0, rows_per_tile // BLK)
        def _(k):
            r = base + k * BLK
            pltpu.make_async_copy(acc.at[pl.ds(r, BLK)],
                                  out_hbm.at[c, pl.ds(r, BLK)], ws).wait()

    return agg


def _norms_m1(xp, degp, w1):
    np_ = xp.shape[0]
    d_hid = w1.shape[1]

    def body(x_ref, dp_ref, w_ref, m_ref, ns_ref, nd_ref):
        dp = dp_ref[...]
        ones = jnp.ones((NW, 1), jnp.float32)
        cn = (((0,), (0,)), ((), ()))
        degs = lax.dot_general(dp[0], ones, cn,
                               preferred_element_type=jnp.float32)
        degd = lax.dot_general(dp[1], ones, cn,
                               preferred_element_type=jnp.float32)
        ns = jnp.where(degs > 0, lax.rsqrt(jnp.maximum(degs, 1.0)), 0.0)
        nd = jnp.where(degd > 0, lax.rsqrt(jnp.maximum(degd, 1.0)), 0.0)
        ns_ref[...] = ns
        nd_ref[...] = nd
        m_ref[...] = jnp.dot(x_ref[...] * ns, w_ref[...],
                             preferred_element_type=jnp.float32)

    return pl.pallas_call(
        body,
        out_shape=(
            jax.ShapeDtypeStruct((np_, d_hid), jnp.float32),
            jax.ShapeDtypeStruct((np_, 1), jnp.float32),
            jax.ShapeDtypeStruct((np_, 1), jnp.float32),
        ),
    )(xp, degp, w1)


def _mid_layer(p, ns, nd, b1, w2):
    np_ = p.shape[1]
    d_out = w2.shape[1]

    def body(p_ref, ns_ref, nd_ref, b_ref, w_ref, m_ref):
        agg = p_ref[0] + p_ref[1]
        h = jnp.maximum(agg * nd_ref[...] + b_ref[...], 0.0)
        m_ref[...] = jnp.dot(h * ns_ref[...], w_ref[...],
                             preferred_element_type=jnp.float32)

    return pl.pallas_call(
        body,
        out_shape=jax.ShapeDtypeStruct((np_, d_out), jnp.float32),
    )(p, ns, nd, b1, w2)


def _head(p, nd, b2, wm1, bm1, wm2, bm2, n):
    d_out = wm2.shape[1]

    def body(p_ref, nd_ref, b_ref, w1_ref, b1_ref, w2_ref, b2_ref, o_ref):
        agg = p_ref[0, :n] + p_ref[1, :n]
        h = jnp.maximum(agg * nd_ref[:n] + b_ref[...], 0.0)
        z = jnp.dot(h, w1_ref[...], preferred_element_type=jnp.float32)
        z = 1.0 / (1.0 + jnp.exp(-(z + b1_ref[...])))
        o_ref[...] = (jnp.dot(z, w2_ref[...],
                              preferred_element_type=jnp.float32) + b2_ref[...])

    return pl.pallas_call(
        body,
        out_shape=jax.ShapeDtypeStruct((n, d_out), jnp.float32),
    )(p, nd, b2, wm1, bm1, wm2, bm2)


def kernel(x, edge_index, W1, b1, W2, b2, Wm1, bm1, Wm2, bm2):
    n, d_in = x.shape
    e = edge_index.shape[1]
    np_, nb = _pad_counts(n, e)
    ep = NW * nb * BLK

    pad_idx = n + (jnp.arange(ep - e, dtype=jnp.int32) % (np_ - n))
    srcp = jnp.concatenate([edge_index[0], pad_idx]).reshape(NW, nb, BLK)
    dstp = jnp.concatenate([edge_index[1], pad_idx]).reshape(NW, nb, BLK)
    xp = jnp.pad(x, ((0, np_ - n), (0, 0)))
    znp = jnp.zeros((np_,), jnp.float32)

    degp = _deg_kernel(np_, nb)(srcp, dstp, znp)
    m1, ns, nd = _norms_m1(xp, degp, W1)
    p1 = _agg_kernel(np_, nb, W1.shape[1], n)(m1, srcp, dstp)
    m2 = _mid_layer(p1, ns, nd, b1.reshape(1, -1), W2)
    p2 = _agg_kernel(np_, nb, W2.shape[1], n)(m2, srcp, dstp)
    return _head(p2, nd, b2.reshape(1, -1), Wm1, bm1.reshape(1, -1),
                 Wm2, bm2.reshape(1, -1), n)


# speedup vs baseline: 1.6713x; 1.0121x over previous
import dataclasses
import functools

import jax
import jax.numpy as jnp
from jax import lax
from jax.experimental import pallas as pl
from jax.experimental.pallas import tpu as pltpu
from jax.experimental.pallas import tpu_sc as plsc

NCORE = 2
NSUB = 16
NW = NCORE * NSUB
BLK = 128
ZR = 64
CH = 40


def _sc_compiler_params():
    cp = pltpu.CompilerParams()
    if "needs_layout_passes" in pltpu.CompilerParams.__dataclass_fields__:
        cp = dataclasses.replace(cp, needs_layout_passes=False)
    return cp


def _pad_counts(n_nodes, n_edges):
    align = NSUB * ZR
    np_ = ((n_nodes + align) // align) * align
    if np_ - n_nodes < BLK:
        np_ += align
    nb = -(-n_edges // (NW * BLK))
    nb = -(-nb // (2 * CH)) * 2 * CH
    return np_, nb


@functools.cache
def _deg_kernel(np_, nb):
    mesh = plsc.VectorSubcoreMesh(core_axis_name="c", subcore_axis_name="s")

    @functools.partial(
        pl.kernel,
        out_type=jax.ShapeDtypeStruct((2, NW, np_), jnp.float32),
        mesh=mesh,
        compiler_params=_sc_compiler_params(),
        scratch_types=[
            pltpu.VMEM((nb, BLK), jnp.int32),
            pltpu.VMEM((nb, BLK), jnp.int32),
            pltpu.VMEM((np_,), jnp.float32),
            pltpu.VMEM((np_,), jnp.float32),
        ],
    )
    def deg(src_hbm, dst_hbm, out_hbm, sidx, didx, hsrc, hdst):
        c = lax.axis_index("c")
        s = lax.axis_index("s")
        w = c * NSUB + s
        pltpu.sync_copy(src_hbm.at[w], sidx)
        pltpu.sync_copy(dst_hbm.at[w], didx)
        zeros = jnp.zeros((16,), jnp.float32)

        @pl.loop(0, np_ // 16)
        def _(i):
            hsrc[pl.ds(i * 16, 16)] = zeros
            hdst[pl.ds(i * 16, 16)] = zeros

        ones = jnp.full((16,), 1.0, jnp.float32)

        @pl.loop(0, nb)
        def _(j):
            for l in range(BLK // 16):
                sv = sidx[j, pl.ds(l * 16, 16)]
                dv = didx[j, pl.ds(l * 16, 16)]
                plsc.addupdate_scatter(hsrc, [sv], ones)
                plsc.addupdate_scatter(hdst, [dv], ones)

        pltpu.sync_copy(hsrc, out_hbm.at[0, w])
        pltpu.sync_copy(hdst, out_hbm.at[1, w])

    return deg


@functools.cache
def _agg_kernel(np_, nb, d, n):
    mesh = plsc.VectorSubcoreMesh(core_axis_name="c", subcore_axis_name="s")
    rows_per_tile = np_ // NSUB

    @functools.partial(
        pl.kernel,
        out_type=jax.ShapeDtypeStruct((NCORE, np_, d), jnp.float32),
        mesh=mesh,
        compiler_params=_sc_compiler_params(),
        scratch_types=[
            pltpu.VMEM((nb, BLK), jnp.int32),
            pltpu.VMEM((CH, BLK), jnp.int32),
            pltpu.VMEM((BLK, d), jnp.float32),
            pltpu.VMEM((BLK, d), jnp.float32),
            pltpu.VMEM((1, BLK), jnp.int32),
            pltpu.VMEM_SHARED((np_, d), jnp.float32),
            pltpu.SemaphoreType.DMA,
            pltpu.SemaphoreType.DMA,
            pltpu.SemaphoreType.DMA,
        ],
    )
    def agg(m_hbm, src_hbm, dst_hbm, out_hbm, sidx, didx, g0, g1, piota,
            acc, s0, s1, ws):
        c = lax.axis_index("c")
        s = lax.axis_index("s")
        w = c * NSUB + s
        pltpu.sync_copy(src_hbm.at[w], sidx)
        base = s * rows_per_tile

        def gstart(j, buf, sem):
            pltpu.async_copy(m_hbm.at[sidx.at[j]], buf, sem)

        def gwait(j, buf, sem):
            pltpu.make_async_copy(m_hbm.at[sidx.at[j]], buf, sem).wait()

        def scat(r, buf):
            pltpu.sync_copy(buf, acc.at[didx.at[r]], add=True)

        gstart(0, g0, s0)
        pltpu.sync_copy(dst_hbm.at[w, pl.ds(0, CH)], didx)

        start = n + s * ((np_ - n - BLK) // NSUB)
        for l in range(BLK // 16):
            piota[0, pl.ds(l * 16, 16)] = (
                start + l * 16 + lax.iota(jnp.int32, 16))
        pltpu.sync_copy(m_hbm.at[piota.at[0]], g1)

        @pl.loop(0, rows_per_tile // BLK)
        def _(k):
            pltpu.sync_copy(g1, acc.at[pl.ds(base + k * BLK, BLK)])

        plsc.subcore_barrier()

        gstart(1, g1, s1)

        @pl.loop(0, nb - 2, step=2)
        def _(j):
            r = lax.rem(j, CH)

            @pl.when(jnp.logical_and(r == 0, j > 0))
            def _():
                pltpu.sync_copy(
                    dst_hbm.at[w, pl.ds(pl.multiple_of(j, CH), CH)], didx)

            gwait(j, g0, s0)
            scat(r, g0)
            gstart(j + 2, g0, s0)
            gwait(j + 1, g1, s1)
            scat(r + 1, g1)

            @pl.when(j + 3 < nb)
            def _():
                gstart(j + 3, g1, s1)

        gwait(nb - 2, g0, s0)
        scat(lax.rem(nb - 2, CH), g0)
        gwait(nb - 1, g1, s1)
        scat(lax.rem(nb - 1, CH), g1)

        plsc.subcore_barrier()

        @pl.loop(0, rows_per_tile // BLK)
        def _(k):
            r = base + k * BLK
            pltpu.async_copy(acc.at[pl.ds(r, BLK)],
                             out_hbm.at[c, pl.ds(r, BLK)], ws)

        @pl.loop(0, rows_per_tile // BLK)
        def _(k):
            r = base + k * BLK
            pltpu.make_async_copy(acc.at[pl.ds(r, BLK)],
                                  out_hbm.at[c, pl.ds(r, BLK)], ws).wait()

    return agg


def _norms_m1(xp, degp, w1):
    np_ = xp.shape[0]
    d_hid = w1.shape[1]

    def body(x_ref, dp_ref, w_ref, m_ref, ns_ref, nd_ref):
        dp = dp_ref[...]
        ones = jnp.ones((NW, 1), jnp.float32)
        cn = (((0,), (0,)), ((), ()))
        degs = lax.dot_general(dp[0], ones, cn,
                               preferred_element_type=jnp.float32)
        degd = lax.dot_general(dp[1], ones, cn,
                               preferred_element_type=jnp.float32)
        ns = jnp.where(degs > 0, lax.rsqrt(jnp.maximum(degs, 1.0)), 0.0)
        nd = jnp.where(degd > 0, lax.rsqrt(jnp.maximum(degd, 1.0)), 0.0)
        ns_ref[...] = ns
        nd_ref[...] = nd
        m_ref[...] = jnp.dot(x_ref[...] * ns, w_ref[...],
                             preferred_element_type=jnp.float32)

    return pl.pallas_call(
        body,
        out_shape=(
            jax.ShapeDtypeStruct((np_, d_hid), jnp.float32),
            jax.ShapeDtypeStruct((np_, 1), jnp.float32),
            jax.ShapeDtypeStruct((np_, 1), jnp.float32),
        ),
    )(xp, degp, w1)


def _mid_layer(p, ns, nd, b1, w2):
    np_ = p.shape[1]
    d_out = w2.shape[1]

    def body(p_ref, ns_ref, nd_ref, b_ref, w_ref, m_ref):
        agg = p_ref[0] + p_ref[1]
        h = jnp.maximum(agg * nd_ref[...] + b_ref[...], 0.0)
        m_ref[...] = jnp.dot(h * ns_ref[...], w_ref[...],
                             preferred_element_type=jnp.float32)

    return pl.pallas_call(
        body,
        out_shape=jax.ShapeDtypeStruct((np_, d_out), jnp.float32),
    )(p, ns, nd, b1, w2)


def _head(p, nd, b2, wm1, bm1, wm2, bm2, n):
    d_out = wm2.shape[1]

    def body(p_ref, nd_ref, b_ref, w1_ref, b1_ref, w2_ref, b2_ref, o_ref):
        agg = p_ref[0, :n] + p_ref[1, :n]
        h = jnp.maximum(agg * nd_ref[:n] + b_ref[...], 0.0)
        z = jnp.dot(h, w1_ref[...], preferred_element_type=jnp.float32)
        z = 1.0 / (1.0 + jnp.exp(-(z + b1_ref[...])))
        o_ref[...] = (jnp.dot(z, w2_ref[...],
                              preferred_element_type=jnp.float32) + b2_ref[...])

    return pl.pallas_call(
        body,
        out_shape=jax.ShapeDtypeStruct((n, d_out), jnp.float32),
    )(p, nd, b2, wm1, bm1, wm2, bm2)


def kernel(x, edge_index, W1, b1, W2, b2, Wm1, bm1, Wm2, bm2):
    n, d_in = x.shape
    e = edge_index.shape[1]
    np_, nb = _pad_counts(n, e)
    ep = NW * nb * BLK

    pad_idx = n + (jnp.arange(ep - e, dtype=jnp.int32) % (np_ - n))
    srcp = jnp.concatenate([edge_index[0], pad_idx]).reshape(NW, nb, BLK)
    dstp = jnp.concatenate([edge_index[1], pad_idx]).reshape(NW, nb, BLK)
    xp = jnp.pad(x, ((0, np_ - n), (0, 0)))
    degp = _deg_kernel(np_, nb)(srcp, dstp)
    m1, ns, nd = _norms_m1(xp, degp, W1)
    p1 = _agg_kernel(np_, nb, W1.shape[1], n)(m1, srcp, dstp)
    m2 = _mid_layer(p1, ns, nd, b1.reshape(1, -1), W2)
    p2 = _agg_kernel(np_, nb, W2.shape[1], n)(m2, srcp, dstp)
    return _head(p2, nd, b2.reshape(1, -1), Wm1, bm1.reshape(1, -1),
                 Wm2, bm2.reshape(1, -1), n)
